# baseline (speedup 1.0000x reference)
import functools

import jax
import jax.numpy as jnp
from jax import lax
from jax.experimental import pallas as pl
from jax.experimental.pallas import tpu as pltpu
from jax.experimental.pallas import tpu_sc as plsc

N = 10000
E = 320000
D = 128
H = 512
L = 64

NC = 4
BN = 1000
NB = N // BN
NTILE = 16
EPT = 20480
EPAD = NTILE * EPT
BROW = 8
BLKE = BROW * 128
BLKS = EPT // BLKE
NROW = 624
NROW_LAST = N - 15 * NROW


def _stage_a_body(x_ref, wg_ref, wgc_ref, asrc_ref, adst_ref, h4_ref,
                  init_ref, scal_ref):
    h = jnp.dot(x_ref[...], wg_ref[...], preferred_element_type=jnp.float32)
    a_s = jnp.dot(h, asrc_ref[...], preferred_element_type=jnp.float32)
    a_d = jnp.dot(h, adst_ref[...], preferred_element_type=jnp.float32)
    t = a_s + a_d
    w_self = jnp.exp(jnp.where(t >= 0, t, 0.2 * t))
    hc = jnp.dot(x_ref[...], wgc_ref[...],
                 preferred_element_type=jnp.float32)
    h4_ref[...] = hc
    init_ref[...] = w_self * hc
    scal_ref[...] = jnp.concatenate([a_s, a_d, w_self], axis=1)


def _stage_a(x, Wg, att_src, att_dst):
    return pl.pallas_call(
        _stage_a_body,
        grid=(NB, NC),
        in_specs=[
            pl.BlockSpec((BN, D), lambda i, c: (i, 0)),
            pl.BlockSpec((D, H), lambda i, c: (0, 0)),
            pl.BlockSpec((D, D), lambda i, c: (0, c)),
            pl.BlockSpec((H, 1), lambda i, c: (0, 0)),
            pl.BlockSpec((H, 1), lambda i, c: (0, 0)),
        ],
        out_specs=[
            pl.BlockSpec((BN, D), lambda i, c: (c * NB + i, 0)),
            pl.BlockSpec((BN, D), lambda i, c: (c * NB + i, 0)),
            pl.BlockSpec((BN, 3), lambda i, c: (i, 0)),
        ],
        out_shape=[
            jax.ShapeDtypeStruct((NC * N, D), jnp.float32),
            jax.ShapeDtypeStruct((NC * N, D), jnp.float32),
            jax.ShapeDtypeStruct((N, 3), jnp.float32),
        ],
    )(x, Wg, Wg, att_src.reshape(H, 1), att_dst.reshape(H, 1))


def _sc_body(h4, init4, a_s_h, a_d_h, wself_h, src_h, dst_h,
             agg_out, den_out,
             src_v, dst_v, gidx_v, a_s_v, a_d_v, w_v, w_f, rows_v,
             agg_sh, den_sh, sem_s, sem_r0, sem_r1):
    c = lax.axis_index("c")
    s = lax.axis_index("s")

    pltpu.sync_copy(a_s_h, a_s_v)
    pltpu.sync_copy(a_d_h, a_d_v)

    for cc in range(2):
        chunk = c + 2 * cc

        srow = pl.multiple_of(s * NROW, 8)
        row0 = pl.multiple_of(chunk * N + s * NROW, 8)

        @pl.when(s < NTILE - 1)
        def _():
            pltpu.sync_copy(init4.at[pl.ds(row0, NROW)],
                            agg_sh.at[pl.ds(srow, NROW)])

        @pl.when(s == NTILE - 1)
        def _():
            pltpu.sync_copy(init4.at[pl.ds(chunk * N + 15 * NROW, NROW_LAST)],
                            agg_sh.at[pl.ds(15 * NROW, NROW_LAST)])
        if cc == 0:
            @pl.when(jnp.logical_and(c == 0, s == 0))
            def _():
                pltpu.sync_copy(wself_h, den_sh)
        plsc.subcore_barrier()

        def block(b, carry):
            pltpu.sync_copy(src_h.at[s, pl.ds(b * BROW, BROW)], src_v)
            pltpu.sync_copy(dst_h.at[s, pl.ds(b * BROW, BROW)], dst_v)

            ebase = s * EPT + b * BLKE
            def weights(j, carry2):
                for g in range(8):
                    sl = pl.ds(g * 16, 16)
                    sv = src_v[j, sl]
                    asv = plsc.load_gather(a_s_v, [sv])
                    adv = plsc.load_gather(a_d_v, [dst_v[j, sl]])
                    t = asv + adv
                    wv = jnp.exp(jnp.where(t >= 0, t, 0.2 * t))
                    pos = (jnp.full((16,), ebase + j * 128 + g * 16,
                                    jnp.int32)
                           + lax.iota(jnp.int32, 16))
                    wv = jnp.where(pos < E, wv, 0.0)
                    w_v[j, sl] = wv
                    w_f[pl.ds(j * 128 + g * 16, 16)] = wv
                    gidx_v[j, sl] = sv + chunk * N
                return carry2
            lax.fori_loop(0, BROW, weights, 0)

            for j in range(BROW):
                pltpu.async_copy(h4.at[gidx_v.at[j]], rows_v,
                                 sem_r0).wait()

                def scale(e, carry2, j=j):
                    ws = plsc.load_gather(
                        w_f, [jnp.full((16,), j * 128 + e, jnp.int32)])
                    for q in range(8):
                        sl = pl.ds(q * 16, 16)
                        rows_v[e, sl] = rows_v[e, sl] * ws
                    return carry2
                lax.fori_loop(0, 128, scale, 0)

                pltpu.sync_copy(rows_v, agg_sh.at[dst_v.at[j]], add=True)
                if cc == 0:
                    @pl.when(c == 0)
                    def _(j=j):
                        pltpu.sync_copy(w_v.at[j],
                                        den_sh.at[dst_v.at[j]], add=True)
            return carry

        lax.fori_loop(0, BLKS, block, 0)
        plsc.subcore_barrier()

        @pl.when(s < NTILE - 1)
        def _():
            pltpu.sync_copy(agg_sh.at[pl.ds(srow, NROW)],
                            agg_out.at[pl.ds(row0, NROW)])

        @pl.when(s == NTILE - 1)
        def _():
            pltpu.sync_copy(agg_sh.at[pl.ds(15 * NROW, NROW_LAST)],
                            agg_out.at[pl.ds(chunk * N + 15 * NROW,
                                             NROW_LAST)])
        if cc == 0:
            @pl.when(jnp.logical_and(c == 0, s == 0))
            def _():
                pltpu.sync_copy(den_sh, den_out)


def _stage_b(h4, init4, a_s, a_d, w_self, srcr, dstr):
    mesh = plsc.VectorSubcoreMesh(core_axis_name="c", subcore_axis_name="s")
    kern = functools.partial(
        pl.kernel,
        out_type=[
            jax.ShapeDtypeStruct((NC * N, D), jnp.float32),
            jax.ShapeDtypeStruct((N,), jnp.float32),
        ],
        mesh=mesh,
        compiler_params=pltpu.CompilerParams(needs_layout_passes=False),
        scratch_types=[
            pltpu.VMEM((BROW, 128), jnp.int32),
            pltpu.VMEM((BROW, 128), jnp.int32),
            pltpu.VMEM((BROW, 128), jnp.int32),
            pltpu.VMEM((N,), jnp.float32),
            pltpu.VMEM((N,), jnp.float32),
            pltpu.VMEM((BROW, 128), jnp.float32),
            pltpu.VMEM((BLKE,), jnp.float32),
            pltpu.VMEM((128, D), jnp.float32),
            pltpu.VMEM_SHARED((N, D), jnp.float32),
            pltpu.VMEM_SHARED((N,), jnp.float32),
            pltpu.SemaphoreType.DMA,
            pltpu.SemaphoreType.DMA,
            pltpu.SemaphoreType.DMA,
        ],
    )(_sc_body)
    return kern(h4, init4, a_s, a_d, w_self, srcr, dstr)


def _stage_c_body(a0, a1, a2, a3, den_ref, x_ref, bg_ref,
                  we1_ref, be1_ref, we2_ref, be2_ref,
                  wd1_ref, bd1_ref, wd2_ref, bd2_ref, wd3_ref, bd3_ref,
                  recon_ref, z_ref, reg_ref, sse_ref):
    i = pl.program_id(0)
    dn = 1.0 / (den_ref[...] + 1e-16)
    aggs = [a0[...], a1[...], a2[...], a3[...]]
    bg = bg_ref[...]
    acc = jnp.zeros((BN, H), jnp.float32)
    for cidx in range(NC):
        hg = aggs[cidx] * dn + bg[:, cidx * D:(cidx + 1) * D]
        h1 = jnp.maximum(hg, 0.0)
        acc = acc + jnp.dot(h1, we1_ref[cidx * D:(cidx + 1) * D, :],
                            preferred_element_type=jnp.float32)
    h2 = jnp.maximum(acc + be1_ref[...], 0.0)
    z = jnp.dot(h2, we2_ref[...], preferred_element_type=jnp.float32) \
        + be2_ref[...]
    d1 = jnp.maximum(jnp.dot(z, wd1_ref[...],
                             preferred_element_type=jnp.float32)
                     + bd1_ref[...], 0.0)
    d2 = jnp.maximum(jnp.dot(d1, wd2_ref[...],
                             preferred_element_type=jnp.float32)
                     + bd2_ref[...], 0.0)
    recon = jnp.dot(d2, wd3_ref[...], preferred_element_type=jnp.float32) \
        + bd3_ref[...]
    recon_ref[...] = recon
    z_ref[...] = z
    reg_ref[...] = jnp.sum(z * z, axis=1, keepdims=True)

    @pl.when(i == 0)
    def _():
        sse_ref[...] = jnp.zeros((1, 1), jnp.float32)
    diff = recon - x_ref[...]
    sse_ref[...] += jnp.sum(diff * diff).reshape(1, 1)


def _stage_c(agg, den, x, bg, We1, be1, We2, be2, Wd1, bd1, Wd2, bd2,
             Wd3, bd3):
    full = lambda r, c: pl.BlockSpec((r, c), lambda i: (0, 0))
    in_specs = [
        pl.BlockSpec((BN, D), lambda i, cc=cc: (cc * NB + i, 0))
        for cc in range(NC)
    ] + [
        pl.BlockSpec((BN, 1), lambda i: (i, 0)),
        pl.BlockSpec((BN, D), lambda i: (i, 0)),
        full(1, H),
        full(H, H), full(1, H),
        full(H, L), full(1, L),
        full(L, H), full(1, H),
        full(H, H), full(1, H),
        full(H, D), full(1, D),
    ]
    out_specs = [
        pl.BlockSpec((BN, D), lambda i: (i, 0)),
        pl.BlockSpec((BN, L), lambda i: (i, 0)),
        pl.BlockSpec((BN, 1), lambda i: (i, 0)),
        pl.BlockSpec((1, 1), lambda i: (0, 0)),
    ]
    out_shape = [
        jax.ShapeDtypeStruct((N, D), jnp.float32),
        jax.ShapeDtypeStruct((N, L), jnp.float32),
        jax.ShapeDtypeStruct((N, 1), jnp.float32),
        jax.ShapeDtypeStruct((1, 1), jnp.float32),
    ]
    return pl.pallas_call(
        _stage_c_body, grid=(NB,), in_specs=in_specs, out_specs=out_specs,
        out_shape=out_shape,
    )(agg, agg, agg, agg, den, x,
      bg.reshape(1, H), We1, be1.reshape(1, H), We2, be2.reshape(1, L),
      Wd1, bd1.reshape(1, H), Wd2, bd2.reshape(1, H), Wd3,
      bd3.reshape(1, D))


def kernel(inputs, edge_index, Wg, att_src, att_dst, bg, We1, be1, We2,
           be2, Wd1, bd1, Wd2, bd2, Wd3, bd3):
    x = inputs[:, 0, :]
    h4, init4, scal = _stage_a(x, Wg, att_src, att_dst)
    a_s = scal[:, 0]
    a_d = scal[:, 1]
    w_self = scal[:, 2]
    ei = edge_index.astype(jnp.int32)
    pad = jnp.zeros((EPAD - E,), jnp.int32)
    srcr = jnp.concatenate([ei[0], pad]).reshape(NTILE, EPT // 128, 128)
    dstr = jnp.concatenate([ei[1], pad]).reshape(NTILE, EPT // 128, 128)
    agg, den = _stage_b(h4, init4, a_s, a_d, w_self, srcr, dstr)
    recon, z, reg2, sse = _stage_c(
        agg, den.reshape(N, 1), x, bg, We1, be1, We2, be2,
        Wd1, bd1, Wd2, bd2, Wd3, bd3)
    reg = reg2[:, 0]
    mse = sse[0, 0] / (N * D)
    total = mse + 0.0 * reg
    return (total, mse, reg, recon, z)


# speedup vs baseline: 1.0664x; 1.0664x over previous
import functools

import jax
import jax.numpy as jnp
from jax import lax
from jax.experimental import pallas as pl
from jax.experimental.pallas import tpu as pltpu
from jax.experimental.pallas import tpu_sc as plsc

N = 10000
E = 320000
D = 128
H = 512
L = 64

NC = 4
BN = 1000
NB = N // BN
NTILE = 16
NWIN = 1280
EPAD = NWIN * 256
KWIN = NWIN // NTILE
NROW = 624
NROW_LAST = N - 15 * NROW


def _stage_a_body(x_ref, wg_ref, wgc_ref, asrc_ref, adst_ref, h4_ref,
                  init_ref, scal_ref):
    h = jnp.dot(x_ref[...], wg_ref[...], preferred_element_type=jnp.float32)
    a_s = jnp.dot(h, asrc_ref[...], preferred_element_type=jnp.float32)
    a_d = jnp.dot(h, adst_ref[...], preferred_element_type=jnp.float32)
    t = a_s + a_d
    w_self = jnp.exp(jnp.where(t >= 0, t, 0.2 * t))
    hc = jnp.dot(x_ref[...], wgc_ref[...],
                 preferred_element_type=jnp.float32)
    h4_ref[...] = hc
    init_ref[...] = w_self * hc
    scal_ref[...] = jnp.concatenate([a_s, a_d, w_self], axis=1)


def _stage_a(x, Wg, att_src, att_dst):
    return pl.pallas_call(
        _stage_a_body,
        grid=(NB, NC),
        in_specs=[
            pl.BlockSpec((BN, D), lambda i, c: (i, 0)),
            pl.BlockSpec((D, H), lambda i, c: (0, 0)),
            pl.BlockSpec((D, D), lambda i, c: (0, c)),
            pl.BlockSpec((H, 1), lambda i, c: (0, 0)),
            pl.BlockSpec((H, 1), lambda i, c: (0, 0)),
        ],
        out_specs=[
            pl.BlockSpec((BN, D), lambda i, c: (c * NB + i, 0)),
            pl.BlockSpec((BN, D), lambda i, c: (c * NB + i, 0)),
            pl.BlockSpec((BN, 3), lambda i, c: (i, 0)),
        ],
        out_shape=[
            jax.ShapeDtypeStruct((NC * N, D), jnp.float32),
            jax.ShapeDtypeStruct((NC * N, D), jnp.float32),
            jax.ShapeDtypeStruct((N, 3), jnp.float32),
        ],
    )(x, Wg, Wg, att_src.reshape(H, 1), att_dst.reshape(H, 1))


def _sc_w_body(a_s_h, a_d_h, wself_h, src_h, dst_h,
               w_out, den_out,
               a_s_v, a_d_v, src_v, dst_v, w_v, den_sh):
    c = lax.axis_index("c")
    s = lax.axis_index("s")

    @pl.when(c == 0)
    def _():
        pltpu.sync_copy(a_s_h, a_s_v)
        pltpu.sync_copy(a_d_h, a_d_v)

        @pl.when(s == 0)
        def _():
            pltpu.sync_copy(wself_h, den_sh)
        plsc.subcore_barrier()

        def window(k, carry):
            w = k * NTILE + s
            pltpu.sync_copy(src_h.at[w], src_v)
            pltpu.sync_copy(dst_h.at[w], dst_v)
            for j in range(2):
                for g in range(8):
                    sl = pl.ds(g * 16, 16)
                    asv = plsc.load_gather(a_s_v, [src_v[j, sl]])
                    adv = plsc.load_gather(a_d_v, [dst_v[j, sl]])
                    t = asv + adv
                    wv = jnp.exp(jnp.where(t >= 0, t, 0.2 * t))
                    pos = (jnp.full((16,), j * 128 + g * 16, jnp.int32)
                           + lax.iota(jnp.int32, 16) + w * 256)
                    w_v[j, sl] = jnp.where(pos < E, wv, 0.0)
            pltpu.sync_copy(w_v, w_out.at[w])
            for j in range(2):
                pltpu.sync_copy(w_v.at[j], den_sh.at[dst_v.at[j]],
                                add=True)
            return carry

        lax.fori_loop(0, KWIN, window, 0)
        plsc.subcore_barrier()

        @pl.when(s == 0)
        def _():
            pltpu.sync_copy(den_sh, den_out)


def _sc_agg_body(h4, init4, src_h, dst_h, w_h,
                 agg_out,
                 src_v, dst_v, gidx_v, w_v, rows2,
                 agg_sh, sem_r0, sem_r1):
    c = lax.axis_index("c")
    s = lax.axis_index("s")
    sem_r = [sem_r0, sem_r1]

    for cc in range(2):
        chunk = c + 2 * cc

        srow = pl.multiple_of(s * NROW, 8)
        row0 = pl.multiple_of(chunk * N + s * NROW, 8)

        @pl.when(s < NTILE - 1)
        def _():
            pltpu.sync_copy(init4.at[pl.ds(row0, NROW)],
                            agg_sh.at[pl.ds(srow, NROW)])

        @pl.when(s == NTILE - 1)
        def _():
            pltpu.sync_copy(init4.at[pl.ds(chunk * N + 15 * NROW, NROW_LAST)],
                            agg_sh.at[pl.ds(15 * NROW, NROW_LAST)])
        plsc.subcore_barrier()

        def window(k, carry):
            w = k * NTILE + s
            pltpu.sync_copy(src_h.at[w], src_v)
            pltpu.sync_copy(dst_h.at[w], dst_v)
            pltpu.sync_copy(w_h.at[w], w_v)
            for j in range(2):
                for g in range(8):
                    sl = pl.ds(g * 16, 16)
                    gidx_v[j, sl] = src_v[j, sl] + chunk * N
            cps = [pltpu.async_copy(h4.at[gidx_v.at[j]], rows2.at[j],
                                    sem_r[j]) for j in range(2)]
            for j in range(2):
                cps[j].wait()

                def scale(e, carry2, j=j):
                    ws = plsc.load_gather(
                        w_v, [jnp.full((16,), j, jnp.int32),
                              jnp.full((16,), e, jnp.int32)])
                    for q in range(8):
                        sl = pl.ds(q * 16, 16)
                        rows2[j, e, sl] = rows2[j, e, sl] * ws
                    return carry2
                lax.fori_loop(0, 128, scale, 0)

                pltpu.sync_copy(rows2.at[j], agg_sh.at[dst_v.at[j]],
                                add=True)
            return carry

        lax.fori_loop(0, KWIN, window, 0)
        plsc.subcore_barrier()

        @pl.when(s < NTILE - 1)
        def _():
            pltpu.sync_copy(agg_sh.at[pl.ds(srow, NROW)],
                            agg_out.at[pl.ds(row0, NROW)])

        @pl.when(s == NTILE - 1)
        def _():
            pltpu.sync_copy(agg_sh.at[pl.ds(15 * NROW, NROW_LAST)],
                            agg_out.at[pl.ds(chunk * N + 15 * NROW,
                                             NROW_LAST)])


def _stage_b(h4, init4, a_s, a_d, w_self, srcr, dstr):
    mesh = plsc.VectorSubcoreMesh(core_axis_name="c", subcore_axis_name="s")
    w_edges, den = functools.partial(
        pl.kernel,
        out_type=[
            jax.ShapeDtypeStruct((NWIN, 2, 128), jnp.float32),
            jax.ShapeDtypeStruct((N,), jnp.float32),
        ],
        mesh=mesh,
        compiler_params=pltpu.CompilerParams(needs_layout_passes=False),
        scratch_types=[
            pltpu.VMEM((N,), jnp.float32),
            pltpu.VMEM((N,), jnp.float32),
            pltpu.VMEM((2, 128), jnp.int32),
            pltpu.VMEM((2, 128), jnp.int32),
            pltpu.VMEM((2, 128), jnp.float32),
            pltpu.VMEM_SHARED((N,), jnp.float32),
        ],
    )(_sc_w_body)(a_s, a_d, w_self, srcr, dstr)

    agg = functools.partial(
        pl.kernel,
        out_type=jax.ShapeDtypeStruct((NC * N, D), jnp.float32),
        mesh=mesh,
        compiler_params=pltpu.CompilerParams(needs_layout_passes=False),
        scratch_types=[
            pltpu.VMEM((2, 128), jnp.int32),
            pltpu.VMEM((2, 128), jnp.int32),
            pltpu.VMEM((2, 128), jnp.int32),
            pltpu.VMEM((2, 128), jnp.float32),
            pltpu.VMEM((2, 128, D), jnp.float32),
            pltpu.VMEM_SHARED((N, D), jnp.float32),
            pltpu.SemaphoreType.DMA,
            pltpu.SemaphoreType.DMA,
        ],
    )(_sc_agg_body)(h4, init4, srcr, dstr, w_edges)
    return agg, den


def _stage_c_body(a0, a1, a2, a3, den_ref, x_ref, bg_ref,
                  we1_ref, be1_ref, we2_ref, be2_ref,
                  wd1_ref, bd1_ref, wd2_ref, bd2_ref, wd3_ref, bd3_ref,
                  recon_ref, z_ref, reg_ref, sse_ref):
    i = pl.program_id(0)
    dn = 1.0 / (den_ref[...] + 1e-16)
    aggs = [a0[...], a1[...], a2[...], a3[...]]
    bg = bg_ref[...]
    acc = jnp.zeros((BN, H), jnp.float32)
    for cidx in range(NC):
        hg = aggs[cidx] * dn + bg[:, cidx * D:(cidx + 1) * D]
        h1 = jnp.maximum(hg, 0.0)
        acc = acc + jnp.dot(h1, we1_ref[cidx * D:(cidx + 1) * D, :],
                            preferred_element_type=jnp.float32)
    h2 = jnp.maximum(acc + be1_ref[...], 0.0)
    z = jnp.dot(h2, we2_ref[...], preferred_element_type=jnp.float32) \
        + be2_ref[...]
    d1 = jnp.maximum(jnp.dot(z, wd1_ref[...],
                             preferred_element_type=jnp.float32)
                     + bd1_ref[...], 0.0)
    d2 = jnp.maximum(jnp.dot(d1, wd2_ref[...],
                             preferred_element_type=jnp.float32)
                     + bd2_ref[...], 0.0)
    recon = jnp.dot(d2, wd3_ref[...], preferred_element_type=jnp.float32) \
        + bd3_ref[...]
    recon_ref[...] = recon
    z_ref[...] = z
    reg_ref[...] = jnp.sum(z * z, axis=1, keepdims=True)

    @pl.when(i == 0)
    def _():
        sse_ref[...] = jnp.zeros((1, 1), jnp.float32)
    diff = recon - x_ref[...]
    sse_ref[...] += jnp.sum(diff * diff).reshape(1, 1)


def _stage_c(agg, den, x, bg, We1, be1, We2, be2, Wd1, bd1, Wd2, bd2,
             Wd3, bd3):
    full = lambda r, c: pl.BlockSpec((r, c), lambda i: (0, 0))
    in_specs = [
        pl.BlockSpec((BN, D), lambda i, cc=cc: (cc * NB + i, 0))
        for cc in range(NC)
    ] + [
        pl.BlockSpec((BN, 1), lambda i: (i, 0)),
        pl.BlockSpec((BN, D), lambda i: (i, 0)),
        full(1, H),
        full(H, H), full(1, H),
        full(H, L), full(1, L),
        full(L, H), full(1, H),
        full(H, H), full(1, H),
        full(H, D), full(1, D),
    ]
    out_specs = [
        pl.BlockSpec((BN, D), lambda i: (i, 0)),
        pl.BlockSpec((BN, L), lambda i: (i, 0)),
        pl.BlockSpec((BN, 1), lambda i: (i, 0)),
        pl.BlockSpec((1, 1), lambda i: (0, 0)),
    ]
    out_shape = [
        jax.ShapeDtypeStruct((N, D), jnp.float32),
        jax.ShapeDtypeStruct((N, L), jnp.float32),
        jax.ShapeDtypeStruct((N, 1), jnp.float32),
        jax.ShapeDtypeStruct((1, 1), jnp.float32),
    ]
    return pl.pallas_call(
        _stage_c_body, grid=(NB,), in_specs=in_specs, out_specs=out_specs,
        out_shape=out_shape,
    )(agg, agg, agg, agg, den, x,
      bg.reshape(1, H), We1, be1.reshape(1, H), We2, be2.reshape(1, L),
      Wd1, bd1.reshape(1, H), Wd2, bd2.reshape(1, H), Wd3,
      bd3.reshape(1, D))


def kernel(inputs, edge_index, Wg, att_src, att_dst, bg, We1, be1, We2,
           be2, Wd1, bd1, Wd2, bd2, Wd3, bd3):
    x = inputs[:, 0, :]
    h4, init4, scal = _stage_a(x, Wg, att_src, att_dst)
    a_s = scal[:, 0]
    a_d = scal[:, 1]
    w_self = scal[:, 2]
    ei = edge_index.astype(jnp.int32)
    pad = jnp.zeros((EPAD - E,), jnp.int32)
    srcr = jnp.concatenate([ei[0], pad]).reshape(NWIN, 2, 128)
    dstr = jnp.concatenate([ei[1], pad]).reshape(NWIN, 2, 128)
    agg, den = _stage_b(h4, init4, a_s, a_d, w_self, srcr, dstr)
    recon, z, reg2, sse = _stage_c(
        agg, den.reshape(N, 1), x, bg, We1, be1, We2, be2,
        Wd1, bd1, Wd2, bd2, Wd3, bd3)
    reg = reg2[:, 0]
    mse = sse[0, 0] / (N * D)
    total = mse + 0.0 * reg
    return (total, mse, reg, recon, z)


# speedup vs baseline: 1.1749x; 1.1017x over previous
import functools

import jax
import jax.numpy as jnp
from jax import lax
from jax.experimental import pallas as pl
from jax.experimental.pallas import tpu as pltpu
from jax.experimental.pallas import tpu_sc as plsc

N = 10000
E = 320000
D = 128
H = 512
L = 64

NC = 4
BN = 1000
NB = N // BN
NTILE = 16
NWIN = 1280
EPAD = NWIN * 256
KWIN = NWIN // NTILE
NROW = 624
NROW_LAST = N - 15 * NROW


def _stage_a_body(x_ref, wg_ref, wgc_ref, asrc_ref, adst_ref, h4_ref,
                  init_ref, scal_ref):
    h = jnp.dot(x_ref[...], wg_ref[...], preferred_element_type=jnp.float32)
    a_s = jnp.dot(h, asrc_ref[...], preferred_element_type=jnp.float32)
    a_d = jnp.dot(h, adst_ref[...], preferred_element_type=jnp.float32)
    t = a_s + a_d
    w_self = jnp.exp(jnp.where(t >= 0, t, 0.2 * t))
    hc = jnp.dot(x_ref[...], wgc_ref[...],
                 preferred_element_type=jnp.float32)
    h4_ref[...] = hc
    init_ref[...] = w_self * hc
    scal_ref[...] = jnp.concatenate([a_s, a_d, w_self], axis=1)


def _stage_a(x, Wg, att_src, att_dst):
    return pl.pallas_call(
        _stage_a_body,
        grid=(NB, NC),
        in_specs=[
            pl.BlockSpec((BN, D), lambda i, c: (i, 0)),
            pl.BlockSpec((D, H), lambda i, c: (0, 0)),
            pl.BlockSpec((D, D), lambda i, c: (0, c)),
            pl.BlockSpec((H, 1), lambda i, c: (0, 0)),
            pl.BlockSpec((H, 1), lambda i, c: (0, 0)),
        ],
        out_specs=[
            pl.BlockSpec((BN, D), lambda i, c: (c * NB + i, 0)),
            pl.BlockSpec((BN, D), lambda i, c: (c * NB + i, 0)),
            pl.BlockSpec((BN, 3), lambda i, c: (i, 0)),
        ],
        out_shape=[
            jax.ShapeDtypeStruct((NC * N, D), jnp.float32),
            jax.ShapeDtypeStruct((NC * N, D), jnp.float32),
            jax.ShapeDtypeStruct((N, 3), jnp.float32),
        ],
    )(x, Wg, Wg, att_src.reshape(H, 1), att_dst.reshape(H, 1))


def _sc_w_body(a_s_h, a_d_h, wself_h, src_h, dst_h,
               w_out, den_out,
               a_s_v, a_d_v, src_v, dst_v, w_v, den_sh):
    c = lax.axis_index("c")
    s = lax.axis_index("s")

    @pl.when(c == 0)
    def _():
        pltpu.sync_copy(a_s_h, a_s_v)
        pltpu.sync_copy(a_d_h, a_d_v)

        @pl.when(s == 0)
        def _():
            pltpu.sync_copy(wself_h, den_sh)
        plsc.subcore_barrier()

        def window(k, carry):
            w = k * NTILE + s
            pltpu.sync_copy(src_h.at[w], src_v)
            pltpu.sync_copy(dst_h.at[w], dst_v)
            for j in range(2):
                for g in range(8):
                    sl = pl.ds(g * 16, 16)
                    asv = plsc.load_gather(a_s_v, [src_v[j, sl]])
                    adv = plsc.load_gather(a_d_v, [dst_v[j, sl]])
                    t = asv + adv
                    wv = jnp.exp(jnp.where(t >= 0, t, 0.2 * t))
                    pos = (jnp.full((16,), j * 128 + g * 16, jnp.int32)
                           + lax.iota(jnp.int32, 16) + w * 256)
                    w_v[j, sl] = jnp.where(pos < E, wv, 0.0)
            pltpu.sync_copy(w_v, w_out.at[w])
            for j in range(2):
                pltpu.sync_copy(w_v.at[j], den_sh.at[dst_v.at[j]],
                                add=True)
            return carry

        lax.fori_loop(0, KWIN, window, 0)
        plsc.subcore_barrier()

        @pl.when(s == 0)
        def _():
            pltpu.sync_copy(den_sh, den_out)


def _sc_agg_body(h4, init4, src_h, dst_h, w_h,
                 agg_out,
                 src_v, dst_v, gidx_v, w_v, rows2,
                 agg_sh, sem_r0, sem_r1):
    c = lax.axis_index("c")
    s = lax.axis_index("s")
    sem_r = [sem_r0, sem_r1]

    for cc in range(2):
        chunk = c + 2 * cc

        srow = pl.multiple_of(s * NROW, 8)
        row0 = pl.multiple_of(chunk * N + s * NROW, 8)

        @pl.when(s < NTILE - 1)
        def _():
            pltpu.sync_copy(init4.at[pl.ds(row0, NROW)],
                            agg_sh.at[pl.ds(srow, NROW)])

        @pl.when(s == NTILE - 1)
        def _():
            pltpu.sync_copy(init4.at[pl.ds(chunk * N + 15 * NROW, NROW_LAST)],
                            agg_sh.at[pl.ds(15 * NROW, NROW_LAST)])
        plsc.subcore_barrier()

        def window(k, carry):
            w = k * NTILE + s
            pltpu.sync_copy(src_h.at[w], src_v)
            pltpu.sync_copy(dst_h.at[w], dst_v)
            pltpu.sync_copy(w_h.at[w], w_v)
            for j in range(2):
                for g in range(8):
                    sl = pl.ds(g * 16, 16)
                    gidx_v[j, sl] = src_v[j, sl] + chunk * N
            cps = [pltpu.async_copy(h4.at[gidx_v.at[j]], rows2.at[j],
                                    sem_r[j]) for j in range(2)]
            for j in range(2):
                cps[j].wait()

                @plsc.parallel_loop(0, 128, unroll=4)
                def _(e, j=j):
                    ws = plsc.load_gather(
                        w_v, [jnp.full((16,), j, jnp.int32),
                              jnp.full((16,), e, jnp.int32)])
                    for q in range(8):
                        sl = pl.ds(q * 16, 16)
                        rows2[j, e, sl] = rows2[j, e, sl] * ws

                pltpu.sync_copy(rows2.at[j], agg_sh.at[dst_v.at[j]],
                                add=True)
            return carry

        lax.fori_loop(0, KWIN, window, 0)
        plsc.subcore_barrier()

        @pl.when(s < NTILE - 1)
        def _():
            pltpu.sync_copy(agg_sh.at[pl.ds(srow, NROW)],
                            agg_out.at[pl.ds(row0, NROW)])

        @pl.when(s == NTILE - 1)
        def _():
            pltpu.sync_copy(agg_sh.at[pl.ds(15 * NROW, NROW_LAST)],
                            agg_out.at[pl.ds(chunk * N + 15 * NROW,
                                             NROW_LAST)])


def _stage_b(h4, init4, a_s, a_d, w_self, srcr, dstr):
    mesh = plsc.VectorSubcoreMesh(core_axis_name="c", subcore_axis_name="s")
    w_edges, den = functools.partial(
        pl.kernel,
        out_type=[
            jax.ShapeDtypeStruct((NWIN, 2, 128), jnp.float32),
            jax.ShapeDtypeStruct((N,), jnp.float32),
        ],
        mesh=mesh,
        compiler_params=pltpu.CompilerParams(needs_layout_passes=False),
        scratch_types=[
            pltpu.VMEM((N,), jnp.float32),
            pltpu.VMEM((N,), jnp.float32),
            pltpu.VMEM((2, 128), jnp.int32),
            pltpu.VMEM((2, 128), jnp.int32),
            pltpu.VMEM((2, 128), jnp.float32),
            pltpu.VMEM_SHARED((N,), jnp.float32),
        ],
    )(_sc_w_body)(a_s, a_d, w_self, srcr, dstr)

    agg = functools.partial(
        pl.kernel,
        out_type=jax.ShapeDtypeStruct((NC * N, D), jnp.float32),
        mesh=mesh,
        compiler_params=pltpu.CompilerParams(needs_layout_passes=False),
        scratch_types=[
            pltpu.VMEM((2, 128), jnp.int32),
            pltpu.VMEM((2, 128), jnp.int32),
            pltpu.VMEM((2, 128), jnp.int32),
            pltpu.VMEM((2, 128), jnp.float32),
            pltpu.VMEM((2, 128, D), jnp.float32),
            pltpu.VMEM_SHARED((N, D), jnp.float32),
            pltpu.SemaphoreType.DMA,
            pltpu.SemaphoreType.DMA,
        ],
    )(_sc_agg_body)(h4, init4, srcr, dstr, w_edges)
    return agg, den


def _stage_c_body(a0, a1, a2, a3, den_ref, x_ref, bg_ref,
                  we1_ref, be1_ref, we2_ref, be2_ref,
                  wd1_ref, bd1_ref, wd2_ref, bd2_ref, wd3_ref, bd3_ref,
                  recon_ref, z_ref, reg_ref, sse_ref):
    i = pl.program_id(0)
    dn = 1.0 / (den_ref[...] + 1e-16)
    aggs = [a0[...], a1[...], a2[...], a3[...]]
    bg = bg_ref[...]
    acc = jnp.zeros((BN, H), jnp.float32)
    for cidx in range(NC):
        hg = aggs[cidx] * dn + bg[:, cidx * D:(cidx + 1) * D]
        h1 = jnp.maximum(hg, 0.0)
        acc = acc + jnp.dot(h1, we1_ref[cidx * D:(cidx + 1) * D, :],
                            preferred_element_type=jnp.float32)
    h2 = jnp.maximum(acc + be1_ref[...], 0.0)
    z = jnp.dot(h2, we2_ref[...], preferred_element_type=jnp.float32) \
        + be2_ref[...]
    d1 = jnp.maximum(jnp.dot(z, wd1_ref[...],
                             preferred_element_type=jnp.float32)
                     + bd1_ref[...], 0.0)
    d2 = jnp.maximum(jnp.dot(d1, wd2_ref[...],
                             preferred_element_type=jnp.float32)
                     + bd2_ref[...], 0.0)
    recon = jnp.dot(d2, wd3_ref[...], preferred_element_type=jnp.float32) \
        + bd3_ref[...]
    recon_ref[...] = recon
    z_ref[...] = z
    reg_ref[...] = jnp.sum(z * z, axis=1, keepdims=True)

    @pl.when(i == 0)
    def _():
        sse_ref[...] = jnp.zeros((1, 1), jnp.float32)
    diff = recon - x_ref[...]
    sse_ref[...] += jnp.sum(diff * diff).reshape(1, 1)


def _stage_c(agg, den, x, bg, We1, be1, We2, be2, Wd1, bd1, Wd2, bd2,
             Wd3, bd3):
    full = lambda r, c: pl.BlockSpec((r, c), lambda i: (0, 0))
    in_specs = [
        pl.BlockSpec((BN, D), lambda i, cc=cc: (cc * NB + i, 0))
        for cc in range(NC)
    ] + [
        pl.BlockSpec((BN, 1), lambda i: (i, 0)),
        pl.BlockSpec((BN, D), lambda i: (i, 0)),
        full(1, H),
        full(H, H), full(1, H),
        full(H, L), full(1, L),
        full(L, H), full(1, H),
        full(H, H), full(1, H),
        full(H, D), full(1, D),
    ]
    out_specs = [
        pl.BlockSpec((BN, D), lambda i: (i, 0)),
        pl.BlockSpec((BN, L), lambda i: (i, 0)),
        pl.BlockSpec((BN, 1), lambda i: (i, 0)),
        pl.BlockSpec((1, 1), lambda i: (0, 0)),
    ]
    out_shape = [
        jax.ShapeDtypeStruct((N, D), jnp.float32),
        jax.ShapeDtypeStruct((N, L), jnp.float32),
        jax.ShapeDtypeStruct((N, 1), jnp.float32),
        jax.ShapeDtypeStruct((1, 1), jnp.float32),
    ]
    return pl.pallas_call(
        _stage_c_body, grid=(NB,), in_specs=in_specs, out_specs=out_specs,
        out_shape=out_shape,
    )(agg, agg, agg, agg, den, x,
      bg.reshape(1, H), We1, be1.reshape(1, H), We2, be2.reshape(1, L),
      Wd1, bd1.reshape(1, H), Wd2, bd2.reshape(1, H), Wd3,
      bd3.reshape(1, D))


def kernel(inputs, edge_index, Wg, att_src, att_dst, bg, We1, be1, We2,
           be2, Wd1, bd1, Wd2, bd2, Wd3, bd3):
    x = inputs[:, 0, :]
    h4, init4, scal = _stage_a(x, Wg, att_src, att_dst)
    a_s = scal[:, 0]
    a_d = scal[:, 1]
    w_self = scal[:, 2]
    ei = edge_index.astype(jnp.int32)
    pad = jnp.zeros((EPAD - E,), jnp.int32)
    srcr = jnp.concatenate([ei[0], pad]).reshape(NWIN, 2, 128)
    dstr = jnp.concatenate([ei[1], pad]).reshape(NWIN, 2, 128)
    agg, den = _stage_b(h4, init4, a_s, a_d, w_self, srcr, dstr)
    recon, z, reg2, sse = _stage_c(
        agg, den.reshape(N, 1), x, bg, We1, be1, We2, be2,
        Wd1, bd1, Wd2, bd2, Wd3, bd3)
    reg = reg2[:, 0]
    mse = sse[0, 0] / (N * D)
    total = mse + 0.0 * reg
    return (total, mse, reg, recon, z)


# speedup vs baseline: 1.7474x; 1.4873x over previous
import functools

import jax
import jax.numpy as jnp
from jax import lax
from jax.experimental import pallas as pl
from jax.experimental.pallas import tpu as pltpu
from jax.experimental.pallas import tpu_sc as plsc

N = 10000
E = 320000
D = 128
H = 512
L = 64

NC = 4
BN = 1000
NB = N // BN
NTILE = 16
WE = 256
NW = E // WE
NROW = 624
NROW_LAST = N - 15 * NROW


def _stage_a_body(x_ref, wg_ref, wgc_ref, asrc_ref, adst_ref, h4_ref,
                  init_ref, scal_ref):
    h = jnp.dot(x_ref[...], wg_ref[...], preferred_element_type=jnp.float32)
    a_s = jnp.dot(h, asrc_ref[...], preferred_element_type=jnp.float32)
    a_d = jnp.dot(h, adst_ref[...], preferred_element_type=jnp.float32)
    t = a_s + a_d
    w_self = jnp.exp(jnp.where(t >= 0, t, 0.2 * t))
    hc = jnp.dot(x_ref[...], wgc_ref[...],
                 preferred_element_type=jnp.float32)
    h4_ref[...] = hc
    init_ref[...] = w_self * hc
    scal_ref[...] = jnp.concatenate([a_s, a_d, w_self], axis=1)


def _stage_a(x, Wg, att_src, att_dst):
    return pl.pallas_call(
        _stage_a_body,
        grid=(NB, NC),
        in_specs=[
            pl.BlockSpec((BN, D), lambda i, c: (i, 0)),
            pl.BlockSpec((D, H), lambda i, c: (0, 0)),
            pl.BlockSpec((D, D), lambda i, c: (0, c)),
            pl.BlockSpec((H, 1), lambda i, c: (0, 0)),
            pl.BlockSpec((H, 1), lambda i, c: (0, 0)),
        ],
        out_specs=[
            pl.BlockSpec((BN, D), lambda i, c: (c * NB + i, 0)),
            pl.BlockSpec((BN, D), lambda i, c: (c * NB + i, 0)),
            pl.BlockSpec((BN, 3), lambda i, c: (i, 0)),
        ],
        out_shape=[
            jax.ShapeDtypeStruct((NC * N, D), jnp.float32),
            jax.ShapeDtypeStruct((NC * N, D), jnp.float32),
            jax.ShapeDtypeStruct((N, 3), jnp.float32),
        ],
    )(x, Wg, Wg, att_src.reshape(H, 1), att_dst.reshape(H, 1))


def _sc_body(h4, init4, a_s_h, a_d_h, wself_h, src_h, dst_h,
             agg_out, den_out,
             src_v, dst_v, gidx_v, a_s_v, a_d_v, w_v, w_f, rows_v,
             agg_sh, den_sh, sem):
    c = lax.axis_index("c")
    s = lax.axis_index("s")

    pltpu.sync_copy(a_s_h, a_s_v)
    pltpu.sync_copy(a_d_h, a_d_v)

    nw_s = jnp.where(s < NW % NTILE, NW // NTILE + 1, NW // NTILE)

    for cc in range(2):
        chunk = c + 2 * cc

        srow = pl.multiple_of(s * NROW, 8)
        row0 = pl.multiple_of(chunk * N + s * NROW, 8)

        @pl.when(s < NTILE - 1)
        def _():
            pltpu.sync_copy(init4.at[pl.ds(row0, NROW)],
                            agg_sh.at[pl.ds(srow, NROW)])

        @pl.when(s == NTILE - 1)
        def _():
            pltpu.sync_copy(init4.at[pl.ds(chunk * N + 15 * NROW, NROW_LAST)],
                            agg_sh.at[pl.ds(15 * NROW, NROW_LAST)])
        if cc == 0:
            @pl.when(jnp.logical_and(c == 0, s == 0))
            def _():
                pltpu.sync_copy(wself_h, den_sh)
        plsc.subcore_barrier()

        def window(k, carry):
            w = k * NTILE + s
            pltpu.sync_copy(src_h.at[w], src_v)
            pltpu.sync_copy(dst_h.at[w], dst_v)
            for j2 in range(2):
                for g2 in range(8):
                    sl = pl.ds(g2 * 16, 16)
                    sv = src_v[j2, sl]
                    dv = dst_v[j2, sl]
                    gidx_v[j2, sl] = sv + chunk * N
                    asv = plsc.load_gather(a_s_v, [sv])
                    adv = plsc.load_gather(a_d_v, [dv])
                    t = asv + adv
                    wv = jnp.exp(jnp.where(t >= 0, t, 0.2 * t))
                    w_v[j2, sl] = wv
                    w_f[sl] = wv
                pltpu.async_copy(h4.at[gidx_v.at[j2]], rows_v, sem).wait()

                @plsc.parallel_loop(0, 128, unroll=4)
                def _(e):
                    ws = plsc.load_gather(
                        w_f, [jnp.full((16,), e, jnp.int32)])
                    for q in range(8):
                        sl = pl.ds(q * 16, 16)
                        rows_v[e, sl] = rows_v[e, sl] * ws

                pltpu.sync_copy(rows_v, agg_sh.at[dst_v.at[j2]], add=True)
                if cc == 0:
                    @pl.when(c == 0)
                    def _():
                        pltpu.sync_copy(w_v.at[j2],
                                        den_sh.at[dst_v.at[j2]], add=True)
            return carry

        lax.fori_loop(0, nw_s, window, 0)
        plsc.subcore_barrier()

        @pl.when(s < NTILE - 1)
        def _():
            pltpu.sync_copy(agg_sh.at[pl.ds(srow, NROW)],
                            agg_out.at[pl.ds(row0, NROW)])

        @pl.when(s == NTILE - 1)
        def _():
            pltpu.sync_copy(agg_sh.at[pl.ds(15 * NROW, NROW_LAST)],
                            agg_out.at[pl.ds(chunk * N + 15 * NROW,
                                             NROW_LAST)])
        if cc == 0:
            @pl.when(jnp.logical_and(c == 0, s == 0))
            def _():
                pltpu.sync_copy(den_sh, den_out)


def _stage_b(h4, init4, a_s, a_d, w_self, srcr, dstr):
    mesh = plsc.VectorSubcoreMesh(core_axis_name="c", subcore_axis_name="s")
    kern = functools.partial(
        pl.kernel,
        out_type=[
            jax.ShapeDtypeStruct((NC * N, D), jnp.float32),
            jax.ShapeDtypeStruct((N,), jnp.float32),
        ],
        mesh=mesh,
        compiler_params=pltpu.CompilerParams(needs_layout_passes=False),
        scratch_types=[
            pltpu.VMEM((2, 128), jnp.int32),
            pltpu.VMEM((2, 128), jnp.int32),
            pltpu.VMEM((2, 128), jnp.int32),
            pltpu.VMEM((N,), jnp.float32),
            pltpu.VMEM((N,), jnp.float32),
            pltpu.VMEM((2, 128), jnp.float32),
            pltpu.VMEM((128,), jnp.float32),
            pltpu.VMEM((128, D), jnp.float32),
            pltpu.VMEM_SHARED((N, D), jnp.float32),
            pltpu.VMEM_SHARED((N,), jnp.float32),
            pltpu.SemaphoreType.DMA,
        ],
    )(_sc_body)
    return kern(h4, init4, a_s, a_d, w_self, srcr, dstr)


def _stage_c_body(a0, a1, a2, a3, den_ref, x_ref, bg_ref,
                  we1_ref, be1_ref, we2_ref, be2_ref,
                  wd1_ref, bd1_ref, wd2_ref, bd2_ref, wd3_ref, bd3_ref,
                  recon_ref, z_ref, reg_ref, sse_ref):
    i = pl.program_id(0)
    dn = 1.0 / (den_ref[...] + 1e-16)
    aggs = [a0[...], a1[...], a2[...], a3[...]]
    bg = bg_ref[...]
    acc = jnp.zeros((BN, H), jnp.float32)
    for cidx in range(NC):
        hg = aggs[cidx] * dn + bg[:, cidx * D:(cidx + 1) * D]
        h1 = jnp.maximum(hg, 0.0)
        acc = acc + jnp.dot(h1, we1_ref[cidx * D:(cidx + 1) * D, :],
                            preferred_element_type=jnp.float32)
    h2 = jnp.maximum(acc + be1_ref[...], 0.0)
    z = jnp.dot(h2, we2_ref[...], preferred_element_type=jnp.float32) \
        + be2_ref[...]
    d1 = jnp.maximum(jnp.dot(z, wd1_ref[...],
                             preferred_element_type=jnp.float32)
                     + bd1_ref[...], 0.0)
    d2 = jnp.maximum(jnp.dot(d1, wd2_ref[...],
                             preferred_element_type=jnp.float32)
                     + bd2_ref[...], 0.0)
    recon = jnp.dot(d2, wd3_ref[...], preferred_element_type=jnp.float32) \
        + bd3_ref[...]
    recon_ref[...] = recon
    z_ref[...] = z
    reg_ref[...] = jnp.sum(z * z, axis=1, keepdims=True)

    @pl.when(i == 0)
    def _():
        sse_ref[...] = jnp.zeros((1, 1), jnp.float32)
    diff = recon - x_ref[...]
    sse_ref[...] += jnp.sum(diff * diff).reshape(1, 1)


def _stage_c(agg, den, x, bg, We1, be1, We2, be2, Wd1, bd1, Wd2, bd2,
             Wd3, bd3):
    full = lambda r, c: pl.BlockSpec((r, c), lambda i: (0, 0))
    in_specs = [
        pl.BlockSpec((BN, D), lambda i, cc=cc: (cc * NB + i, 0))
        for cc in range(NC)
    ] + [
        pl.BlockSpec((BN, 1), lambda i: (i, 0)),
        pl.BlockSpec((BN, D), lambda i: (i, 0)),
        full(1, H),
        full(H, H), full(1, H),
        full(H, L), full(1, L),
        full(L, H), full(1, H),
        full(H, H), full(1, H),
        full(H, D), full(1, D),
    ]
    out_specs = [
        pl.BlockSpec((BN, D), lambda i: (i, 0)),
        pl.BlockSpec((BN, L), lambda i: (i, 0)),
        pl.BlockSpec((BN, 1), lambda i: (i, 0)),
        pl.BlockSpec((1, 1), lambda i: (0, 0)),
    ]
    out_shape = [
        jax.ShapeDtypeStruct((N, D), jnp.float32),
        jax.ShapeDtypeStruct((N, L), jnp.float32),
        jax.ShapeDtypeStruct((N, 1), jnp.float32),
        jax.ShapeDtypeStruct((1, 1), jnp.float32),
    ]
    return pl.pallas_call(
        _stage_c_body, grid=(NB,), in_specs=in_specs, out_specs=out_specs,
        out_shape=out_shape,
    )(agg, agg, agg, agg, den, x,
      bg.reshape(1, H), We1, be1.reshape(1, H), We2, be2.reshape(1, L),
      Wd1, bd1.reshape(1, H), Wd2, bd2.reshape(1, H), Wd3,
      bd3.reshape(1, D))


def kernel(inputs, edge_index, Wg, att_src, att_dst, bg, We1, be1, We2,
           be2, Wd1, bd1, Wd2, bd2, Wd3, bd3):
    x = inputs[:, 0, :]
    h4, init4, scal = _stage_a(x, Wg, att_src, att_dst)
    a_s = scal[:, 0]
    a_d = scal[:, 1]
    w_self = scal[:, 2]
    ei = edge_index.astype(jnp.int32)
    srcr = ei[0].reshape(NW, 2, 128)
    dstr = ei[1].reshape(NW, 2, 128)
    agg, den = _stage_b(h4, init4, a_s, a_d, w_self, srcr, dstr)
    recon, z, reg2, sse = _stage_c(
        agg, den.reshape(N, 1), x, bg, We1, be1, We2, be2,
        Wd1, bd1, Wd2, bd2, Wd3, bd3)
    reg = reg2[:, 0]
    mse = sse[0, 0] / (N * D)
    total = mse + 0.0 * reg
    return (total, mse, reg, recon, z)
